# baseline (speedup 1.0000x reference)
import functools

import jax
import jax.numpy as jnp
from jax import lax
from jax.experimental import pallas as pl
from jax.experimental.pallas import tpu as pltpu
from jax.experimental.pallas import tpu_sc as plsc

_D = 64
_P = 50
_H = 2
_BATCH = 4096
_NC = 2
_NS = 16
_NW = _NC * _NS
_NB = _BATCH // _NW
_PJ = _P * _H

_NE = 1000000
_NEB = _NE // 128
_NTAIL = _NE - _NEB * 128
_ATRIPS = (_NEB + _NW - 1) // _NW


def _transpose_body(tabt_hbm, tail_hbm, pad_hbm, in_v, out_v, tail_v, gsem, wsem):
    wid = lax.axis_index("s") * _NC + lax.axis_index("c")
    iota = lax.iota(jnp.int32, 16)
    perms = [(iota + k) & 15 for k in range(16)]

    def blk(c):
        return c * _NW + wid

    def e0_of(c):
        return pl.multiple_of(blk(c) * 128, 128)

    def start_gather(c, s):
        return pltpu.async_copy(
            tabt_hbm.at[:, pl.ds(e0_of(c), 128)], in_v.at[s], gsem.at[s]
        )

    def start_write(c, s):
        return pltpu.async_copy(
            out_v.at[s], pad_hbm.at[pl.ds(e0_of(c), 128)], wsem.at[s]
        )

    start_gather(0, 0)

    def step(c, carry):
        s = c & 1
        valid = blk(c) < _NEB

        @pl.when(jnp.logical_and(c >= 2, blk(c - 2) < _NEB))
        def _():
            pltpu.make_async_copy(
                out_v.at[s], pad_hbm.at[pl.ds(0, 128)], wsem.at[s]
            ).wait()

        @pl.when(jnp.logical_and(c + 1 < _ATRIPS, blk(c + 1) < _NEB))
        def _():
            start_gather(c + 1, 1 - s)

        def do_block(ss):
            pltpu.make_async_copy(
                tabt_hbm.at[:, pl.ds(0, 128)], in_v.at[ss], gsem.at[ss]
            ).wait()

            @plsc.parallel_loop(0, 8, unroll=2)
            def _eblock(t):
                ev = iota + t * 16
                for k in range(16):
                    pk = perms[k]
                    for dd in range(4):
                        dv = pk + (dd * 16)
                        vals = plsc.load_gather(in_v.at[ss], [dv, ev])
                        plsc.store_scatter(out_v.at[ss], [ev, dv], vals)

            start_write(c, ss)

        @pl.when(jnp.logical_and(valid, s == 0))
        def _():
            do_block(0)

        @pl.when(jnp.logical_and(valid, s == 1))
        def _():
            do_block(1)

        return carry

    lax.fori_loop(0, _ATRIPS, step, 0)

    for cl in (_ATRIPS - 2, _ATRIPS - 1):
        @pl.when(blk(cl) < _NEB)
        def _():
            pltpu.make_async_copy(
                out_v.at[cl & 1], pad_hbm.at[pl.ds(0, 128)], wsem.at[cl & 1]
            ).wait()

    @pl.when(wid == 0)
    def _():
        pltpu.sync_copy(tail_hbm, tail_v)

        @plsc.parallel_loop(0, _NTAIL // 16)
        def _tailblock(t):
            ev = iota + t * 16
            for k in range(16):
                pk = perms[k]
                for dd in range(4):
                    dv = pk + (dd * 16)
                    vals = plsc.load_gather(tail_v, [dv, ev])
                    plsc.store_scatter(out_v.at[0], [ev, dv], vals)

        pltpu.sync_copy(
            out_v.at[0].at[pl.ds(0, _NTAIL)],
            pad_hbm.at[pl.ds(_NEB * 128, _NTAIL)],
        )


@jax.jit
def _table_transpose(tabt, tail):
    mesh = plsc.VectorSubcoreMesh(core_axis_name="c", subcore_axis_name="s")
    run = pl.kernel(
        _transpose_body,
        out_type=jax.ShapeDtypeStruct((_NE, 128), jnp.float32),
        mesh=mesh,
        scratch_types=[
            pltpu.VMEM((2, _D, 128), jnp.float32),
            pltpu.VMEM((2, 128, 128), jnp.float32),
            pltpu.VMEM((_D, _NTAIL), jnp.float32),
            pltpu.SemaphoreType.DMA((2,)),
            pltpu.SemaphoreType.DMA((2,)),
        ],
        compiler_params=pltpu.CompilerParams(
            use_tc_tiling_on_sc=True,
            needs_layout_passes=False,
            disable_bounds_checks=True,
        ),
    )
    return run(tabt, tail)


def _gather_body(idx_hbm, table_hbm, out_hbm, idx_v, rows_v, outb_v, gsem, wsem):
    wid = lax.axis_index("s") * _NC + lax.axis_index("c")
    pltpu.sync_copy(idx_hbm.at[:, wid], idx_v)

    iota = lax.iota(jnp.int32, 16)
    perms = [(iota + k) & 15 for k in range(16)]

    def start_gather(c, s):
        p = c // _H
        j = c - p * _H
        return pltpu.async_copy(
            table_hbm.at[idx_v.at[p, j]], rows_v.at[s], gsem.at[s]
        )

    def start_write(c, s):
        p = c // _H
        j = c - p * _H
        return pltpu.async_copy(
            outb_v.at[s], out_hbm.at[p, j, :, wid], wsem.at[s]
        )

    start_gather(0, 0)

    def chunk(c, carry):
        s = c & 1

        @pl.when(c >= 2)
        def _():
            pltpu.make_async_copy(
                outb_v.at[s], out_hbm.at[0, 0, :, wid], wsem.at[s]
            ).wait()

        @pl.when(c + 1 < _PJ)
        def _():
            start_gather(c + 1, 1 - s)

        def do_chunk(ss):
            pltpu.make_async_copy(
                table_hbm.at[idx_v.at[0, 0]], rows_v.at[ss], gsem.at[ss]
            ).wait()

            @plsc.parallel_loop(0, _D // 16, unroll=2)
            def _dblock(t):
                d0 = t * 16
                dt0 = t * 2
                for k in range(16):
                    pk = perms[k]
                    dv = pk + d0
                    dtv = (pk >> 3) + dt0
                    dlv = pk & 7
                    for bb in range(_NB // 16):
                        bv = iota + (bb * 16)
                        vals = plsc.load_gather(rows_v.at[ss], [bv, dv])
                        plsc.store_scatter(outb_v.at[ss], [dtv, dlv, bv], vals)

            start_write(c, ss)

        @pl.when(s == 0)
        def _():
            do_chunk(0)

        @pl.when(s == 1)
        def _():
            do_chunk(1)

        return carry

    lax.fori_loop(0, _PJ, chunk, 0)

    for s in (0, 1):
        pltpu.make_async_copy(
            outb_v.at[s], out_hbm.at[0, 0, :, wid], wsem.at[s]
        ).wait()


@jax.jit
def _embed_lookup(idx_r, table):
    mesh = plsc.VectorSubcoreMesh(core_axis_name="c", subcore_axis_name="s")
    run = pl.kernel(
        _gather_body,
        out_type=jax.ShapeDtypeStruct(
            (_P, _H, _D // 8, _NW, 8, _NB), jnp.float32
        ),
        mesh=mesh,
        scratch_types=[
            pltpu.VMEM((_P, _H, _NB), jnp.int32),
            pltpu.VMEM((2, _NB, 2 * _D), jnp.float32),
            pltpu.VMEM((2, _D // 8, 8, _NB), jnp.float32),
            pltpu.SemaphoreType.DMA((2,)),
            pltpu.SemaphoreType.DMA((2,)),
        ],
        compiler_params=pltpu.CompilerParams(
            use_tc_tiling_on_sc=False,
            needs_layout_passes=False,
            disable_bounds_checks=True,
        ),
    )
    return run(idx_r, table)


def kernel(idx, embedding_weight):
    tabt = jnp.transpose(embedding_weight)
    tab_p = _table_transpose(tabt, lax.slice(tabt, (0, _NEB * 128), (_D, _NE)))
    idx_r = (
        jnp.transpose(idx, (1, 2, 0))
        .reshape(_P, _H, _NW, _NB)
        .transpose(0, 2, 1, 3)
    )
    out_r = _embed_lookup(idx_r, tab_p)
    out = jnp.transpose(out_r, (3, 5, 0, 1, 2, 4)).reshape(
        _BATCH, _P, _H, _D
    )
    return out


# speedup vs baseline: 1.3903x; 1.3903x over previous
import functools

import jax
import jax.numpy as jnp
from jax import lax
from jax.experimental import pallas as pl
from jax.experimental.pallas import tpu as pltpu
from jax.experimental.pallas import tpu_sc as plsc

_D = 64
_P = 50
_H = 2
_BATCH = 4096
_NC = 2
_NS = 16
_NW = _NC * _NS
_NB = _BATCH // _NW
_PJ = _P * _H

_NE = 1000000
_NEB = _NE // 128
_NTAIL = _NE - _NEB * 128
_ATRIPS = (_NEB + _NW - 1) // _NW


def _transpose_body(tabt_hbm, tail_hbm, pad_hbm, in_v, out_v, tail_v, gsem, wsem):
    wid = lax.axis_index("s") * _NC + lax.axis_index("c")
    iota = lax.iota(jnp.int32, 16)
    perms = [(iota + k) & 15 for k in range(16)]

    def blk(c):
        return c * _NW + wid

    def e0_of(c):
        return pl.multiple_of(blk(c) * 128, 128)

    def start_gather(c, s):
        return pltpu.async_copy(
            tabt_hbm.at[:, pl.ds(e0_of(c), 128)], in_v.at[s], gsem.at[s]
        )

    def start_write(c, s):
        return pltpu.async_copy(
            out_v.at[s], pad_hbm.at[pl.ds(e0_of(c), 128)], wsem.at[s]
        )

    start_gather(0, 0)

    def step(c, carry):
        s = c & 1
        valid = blk(c) < _NEB

        @pl.when(jnp.logical_and(c >= 2, blk(c - 2) < _NEB))
        def _():
            pltpu.make_async_copy(
                out_v.at[s], pad_hbm.at[pl.ds(0, 128)], wsem.at[s]
            ).wait()

        @pl.when(jnp.logical_and(c + 1 < _ATRIPS, blk(c + 1) < _NEB))
        def _():
            start_gather(c + 1, 1 - s)

        @pl.when(valid)
        def _():
            pltpu.make_async_copy(
                tabt_hbm.at[:, pl.ds(0, 128)], in_v.at[s], gsem.at[s]
            ).wait()

            @plsc.parallel_loop(0, 8, unroll=2)
            def _eblock(t):
                ev = iota + t * 16
                for k in range(16):
                    pk = perms[k]
                    for dd in range(4):
                        dv = pk + (dd * 16)
                        vals = plsc.load_gather(in_v.at[s], [dv, ev])
                        plsc.store_scatter(out_v.at[s], [ev, dv], vals)

            start_write(c, s)

        return carry

    lax.fori_loop(0, _ATRIPS, step, 0)

    for cl in (_ATRIPS - 2, _ATRIPS - 1):
        @pl.when(blk(cl) < _NEB)
        def _():
            pltpu.make_async_copy(
                out_v.at[cl & 1], pad_hbm.at[pl.ds(0, 128)], wsem.at[cl & 1]
            ).wait()

    @pl.when(wid == 0)
    def _():
        pltpu.sync_copy(tail_hbm, tail_v)

        @plsc.parallel_loop(0, _NTAIL // 16)
        def _tailblock(t):
            ev = iota + t * 16
            for k in range(16):
                pk = perms[k]
                for dd in range(4):
                    dv = pk + (dd * 16)
                    vals = plsc.load_gather(tail_v, [dv, ev])
                    plsc.store_scatter(out_v.at[0], [ev, dv], vals)

        pltpu.sync_copy(
            out_v.at[0].at[pl.ds(0, _NTAIL)],
            pad_hbm.at[pl.ds(_NEB * 128, _NTAIL)],
        )


@jax.jit
def _table_transpose(tabt, tail):
    mesh = plsc.VectorSubcoreMesh(core_axis_name="c", subcore_axis_name="s")
    run = pl.kernel(
        _transpose_body,
        out_type=jax.ShapeDtypeStruct((_NE, 128), jnp.float32),
        mesh=mesh,
        scratch_types=[
            pltpu.VMEM((2, _D, 128), jnp.float32),
            pltpu.VMEM((2, 128, 128), jnp.float32),
            pltpu.VMEM((_D, _NTAIL), jnp.float32),
            pltpu.SemaphoreType.DMA((2,)),
            pltpu.SemaphoreType.DMA((2,)),
        ],
        compiler_params=pltpu.CompilerParams(
            use_tc_tiling_on_sc=True,
            needs_layout_passes=False,
            disable_bounds_checks=True,
        ),
    )
    return run(tabt, tail)


def _gather_body(idx_hbm, table_hbm, out_hbm, idx_v, rows_v, outb_v, gsem, wsem):
    wid = lax.axis_index("s") * _NC + lax.axis_index("c")
    pltpu.sync_copy(idx_hbm.at[:, wid], idx_v)

    iota = lax.iota(jnp.int32, 16)
    perms = [(iota + k) & 15 for k in range(16)]

    def start_gather(c, s):
        p = c // _H
        j = c - p * _H
        return pltpu.async_copy(
            table_hbm.at[idx_v.at[p, j]], rows_v.at[s], gsem.at[s]
        )

    def start_write(c, s):
        p = c // _H
        j = c - p * _H
        return pltpu.async_copy(
            outb_v.at[s], out_hbm.at[p, j, :, wid], wsem.at[s]
        )

    start_gather(0, 0)

    def chunk(c, carry):
        s = c & 1

        @pl.when(c >= 2)
        def _():
            pltpu.make_async_copy(
                outb_v.at[s], out_hbm.at[0, 0, :, wid], wsem.at[s]
            ).wait()

        @pl.when(c + 1 < _PJ)
        def _():
            start_gather(c + 1, 1 - s)

        pltpu.make_async_copy(
            table_hbm.at[idx_v.at[0, 0]], rows_v.at[s], gsem.at[s]
        ).wait()

        @plsc.parallel_loop(0, _D // 16, unroll=2)
        def _dblock(t):
            d0 = t * 16
            dt0 = t * 2
            for k in range(16):
                pk = perms[k]
                dv = pk + d0
                dtv = (pk >> 3) + dt0
                dlv = pk & 7
                for bb in range(_NB // 16):
                    bv = iota + (bb * 16)
                    vals = plsc.load_gather(rows_v.at[s], [bv, dv])
                    plsc.store_scatter(outb_v.at[s], [dtv, dlv, bv], vals)

        start_write(c, s)
        return carry

    lax.fori_loop(0, _PJ, chunk, 0)

    for s in (0, 1):
        pltpu.make_async_copy(
            outb_v.at[s], out_hbm.at[0, 0, :, wid], wsem.at[s]
        ).wait()


@jax.jit
def _embed_lookup(idx_r, table):
    mesh = plsc.VectorSubcoreMesh(core_axis_name="c", subcore_axis_name="s")
    run = pl.kernel(
        _gather_body,
        out_type=jax.ShapeDtypeStruct(
            (_P, _H, _D // 8, _NW, 8, _NB), jnp.float32
        ),
        mesh=mesh,
        scratch_types=[
            pltpu.VMEM((_P, _H, _NB), jnp.int32),
            pltpu.VMEM((2, _NB, 2 * _D), jnp.float32),
            pltpu.VMEM((2, _D // 8, 8, _NB), jnp.float32),
            pltpu.SemaphoreType.DMA((2,)),
            pltpu.SemaphoreType.DMA((2,)),
        ],
        compiler_params=pltpu.CompilerParams(
            use_tc_tiling_on_sc=False,
            needs_layout_passes=False,
            disable_bounds_checks=True,
        ),
    )
    return run(idx_r, table)


def kernel(idx, embedding_weight):
    tabt = jnp.transpose(embedding_weight)
    tab_p = _table_transpose(tabt, lax.slice(tabt, (0, _NEB * 128), (_D, _NE)))
    idx_r = (
        jnp.transpose(idx, (1, 2, 0))
        .reshape(_P, _H, _NW, _NB)
        .transpose(0, 2, 1, 3)
    )
    out_r = _embed_lookup(idx_r, tab_p)
    out = jnp.transpose(out_r, (3, 5, 0, 1, 2, 4)).reshape(
        _BATCH, _P, _H, _D
    )
    return out
